# baseline (speedup 1.0000x reference)
import functools

import jax
import jax.numpy as jnp
from jax import lax
from jax.experimental import pallas as pl
from jax.experimental.pallas import tpu as pltpu
from jax.experimental.pallas import tpu_sc as plsc

_NUM_CLASSES = 1000000
_FEAT = 64
_BATCH = 16384
_LAMBDA_C = 0.001

_NC = 2
_NS = 16
_NW = _NC * _NS
_BPW = _BATCH // _NW
_EPW = _BPW * _FEAT
_CH = 128
_NCH = _EPW // _CH
_GRP = 8
_L = 16


def _partials_kernel(feat_hbm, idx_hbm, cent_hbm, out_hbm,
                     idx_v, gath_v, feat_v, acc_v, sem, fsem):
    wid = lax.axis_index("s") * _NC + lax.axis_index("c")

    pltpu.sync_copy(idx_hbm.at[wid], idx_v)
    fcopy = pltpu.async_copy(feat_hbm.at[pl.ds(wid * _EPW, _EPW)], feat_v,
                             fsem)

    def wave(g, _):
        copies = [
            pltpu.async_copy(
                cent_hbm.at[idx_v.at[g * _GRP + u]],
                gath_v.at[pl.ds((g * _GRP + u) * _CH, _CH)],
                sem)
            for u in range(_GRP)
        ]
        for cp in copies:
            cp.wait()
        return 0

    lax.fori_loop(0, _NCH // _GRP, wave, 0)
    fcopy.wait()

    def body(k, accs):
        accs = list(accs)
        for u in range(4):
            j = k * 4 + u
            x = feat_v[pl.ds(j * _L, _L)]
            y = gath_v[pl.ds(j * _L, _L)]
            d = x - y
            accs[u] += d * d
        return tuple(accs)

    zero = jnp.zeros((_L,), jnp.float32)
    accs = lax.fori_loop(0, _EPW // _L // 4, body, (zero,) * 4)
    acc_v[...] = (accs[0] + accs[1]) + (accs[2] + accs[3])
    pltpu.sync_copy(acc_v, out_hbm.at[wid])


@functools.partial(
    pl.kernel,
    mesh=plsc.VectorSubcoreMesh(core_axis_name="c", subcore_axis_name="s"),
    out_type=jax.ShapeDtypeStruct((_NW, _L), jnp.float32),
    scratch_types=[
        pltpu.VMEM((_NCH, _CH), jnp.int32),
        pltpu.VMEM((_EPW,), jnp.float32),
        pltpu.VMEM((_EPW,), jnp.float32),
        pltpu.VMEM((_L,), jnp.float32),
        pltpu.SemaphoreType.DMA,
        pltpu.SemaphoreType.DMA,
    ],
    compiler_params=pltpu.CompilerParams(use_tc_tiling_on_sc=False),
)
def _partials(feat_hbm, idx_hbm, cent_hbm, out_hbm,
              idx_v, gath_v, feat_v, acc_v, sem, fsem):
    _partials_kernel(feat_hbm, idx_hbm, cent_hbm, out_hbm,
                     idx_v, gath_v, feat_v, acc_v, sem, fsem)


def kernel(features, labels, centers):
    cent_flat = centers.T.reshape(-1)
    feat_flat = features.reshape(-1)
    lab = labels.astype(jnp.int32)
    phys = (lab[:, None] + jnp.arange(_FEAT, dtype=jnp.int32)[None, :]
            * _NUM_CLASSES)
    idx3 = phys.reshape(_NW, _NCH, _CH)
    partials = _partials(feat_flat, idx3, cent_flat)
    return (_LAMBDA_C * 0.5 / _BATCH) * jnp.sum(partials)
